# baseline (speedup 1.0000x reference)
import functools

import jax
import jax.numpy as jnp
import numpy as np
from jax import lax
from jax.experimental import pallas as pl
from jax.experimental.pallas import tpu as pltpu
from jax.experimental.pallas import tpu_sc as plsc

B, T, F = 16, 4096, 256
ROWS = B * T
_P = 0.1


def _threefry2x32(k0, k1, x0, x1):
    def rotl(x, d):
        return ((x << np.uint32(d)) | (x >> np.uint32(32 - d))).astype(np.uint32)
    ks = [np.uint32(k0), np.uint32(k1),
          np.uint32(np.uint32(0x1BD11BDA) ^ np.uint32(k0) ^ np.uint32(k1))]
    rotations = [(13, 15, 26, 6), (17, 29, 16, 24)]
    x0 = (x0 + ks[0]).astype(np.uint32)
    x1 = (x1 + ks[1]).astype(np.uint32)
    for i in range(5):
        for r in rotations[i % 2]:
            x0 = (x0 + x1).astype(np.uint32)
            x1 = rotl(x1, r) ^ x0
        x0 = (x0 + ks[(i + 1) % 3]).astype(np.uint32)
        x1 = (x1 + ks[(i + 2) % 3] + np.uint32(i + 1)).astype(np.uint32)
    return x0, x1


def _uniform_bits(key, n):
    b1, b2 = _threefry2x32(key[0], key[1], np.zeros(n, np.uint32),
                           np.arange(n, dtype=np.uint32))
    bits = b1 ^ b2
    f = ((bits >> np.uint32(9)) | np.uint32(0x3F800000)).view(np.float32)
    return np.maximum(np.float32(0.0), f - np.float32(1.0))


def _compute_perm() -> np.ndarray:
    n = B * T * F
    b1, b2 = _threefry2x32(np.uint32(0), np.uint32(0),
                           np.zeros(2, np.uint32), np.arange(2, dtype=np.uint32))
    k1, k2 = (b1[0], b2[0]), (b1[1], b2[1])
    swap = _uniform_bits(k1, n) < np.float32(_P)
    keys = _uniform_bits(k2, n)
    base = np.broadcast_to(np.arange(F, dtype=np.float32), (B * T, F))
    scores = np.where(swap.reshape(B * T, F), keys.reshape(B * T, F), base)
    return np.argsort(scores, axis=-1, kind="stable").astype(np.int32)


_INFO = plsc.get_sparse_core_info()
_NC = _INFO.num_cores
_NS = _INFO.num_subcores
_L = _INFO.num_lanes
_NW = _NC * _NS
_RPW = ROWS // _NW
_RCH = 32
_NCHUNK = _RPW // _RCH
_CHW = _RCH * F

_PERM = _compute_perm()
_POS = np.empty(F, np.int64)
for _q in range(4):
    for _i in range(16):
        for _j in range(4):
            _POS[64 * _q + 4 * _i + _j] = 16 * (4 * _q + _j) + _i
_FLAT_IDX = _PERM[:, _POS].astype(np.uint8).reshape(ROWS * F).view(np.int32)

_mesh = plsc.VectorSubcoreMesh(core_axis_name="c", subcore_axis_name="s")


@functools.partial(
    pl.kernel,
    mesh=_mesh,
    out_type=jax.ShapeDtypeStruct((ROWS, F), jnp.float32),
    scratch_types=[
        pltpu.VMEM((_RCH, F), jnp.float32),
        pltpu.VMEM((_RCH, F), jnp.float32),
        pltpu.VMEM((_CHW // 4,), jnp.int32),
        pltpu.VMEM((_CHW // 4,), jnp.int32),
        pltpu.VMEM((_RCH, F), jnp.float32),
        pltpu.VMEM((_RCH, F), jnp.float32),
        pltpu.SemaphoreType.DMA((2,)),
        pltpu.SemaphoreType.DMA((2,)),
    ],
    compiler_params=pltpu.CompilerParams(needs_layout_passes=False),
)
def _sc_gather(x_hbm, idx_hbm, out_hbm, xv0, xv1, iv0, iv1, ov0, ov1,
               in_sem, out_sem):
    wid = lax.axis_index("s") * _NC + lax.axis_index("c")
    rbase = wid * _RPW
    ibase = wid * _RPW * (F // 4)
    xv = (xv0, xv1)
    iv = (iv0, iv1)
    ov = (ov0, ov1)

    def start_in(c, b):
        pltpu.async_copy(x_hbm.at[pl.ds(rbase + c * _RCH, _RCH)], xv[b],
                         in_sem.at[b])
        pltpu.async_copy(idx_hbm.at[pl.ds(ibase + c * (_CHW // 4), _CHW // 4)],
                         iv[b], in_sem.at[b])

    def wait_in(b):
        pltpu.make_async_copy(x_hbm.at[pl.ds(0, _RCH)], xv[b],
                              in_sem.at[b]).wait()
        pltpu.make_async_copy(idx_hbm.at[pl.ds(0, _CHW // 4)], iv[b],
                              in_sem.at[b]).wait()

    def wait_out(b):
        pltpu.make_async_copy(ov[b], out_hbm.at[pl.ds(rbase, _RCH)],
                              out_sem.at[b]).wait()

    start_in(0, 0)
    start_in(1, 1)

    def pair(g, carry):
        for b in range(2):
            c = 2 * g + b
            wait_in(b)

            @pl.when(g > 0)
            def _():
                wait_out(b)

            for r in range(_RCH):
                rvec = jnp.full((_L,), r, jnp.int32)
                for q in range(4):
                    w = iv[b][pl.ds(r * (F // 4) + q * _L, _L)]
                    for j in range(4):
                        col = (w >> (8 * j)) & 255
                        vec = plsc.load_gather(xv[b], [rvec, col])
                        ov[b][r, pl.ds((q * 4 + j) * _L, _L)] = vec
            pltpu.async_copy(ov[b], out_hbm.at[pl.ds(rbase + c * _RCH, _RCH)],
                             out_sem.at[b])

            @pl.when(c + 2 < _NCHUNK)
            def _():
                start_in(c + 2, b)

        return carry

    lax.fori_loop(0, _NCHUNK // 2, pair, 0)
    for b in range(2):
        wait_out(b)


def kernel(x):
    out = _sc_gather(x.reshape(ROWS, F), jnp.asarray(_FLAT_IDX))
    return out.reshape(B, T, F)


# speedup vs baseline: 1.8507x; 1.8507x over previous
import functools

import jax
import jax.numpy as jnp
import numpy as np
from jax import lax
from jax.experimental import pallas as pl
from jax.experimental.pallas import tpu as pltpu
from jax.experimental.pallas import tpu_sc as plsc

B, T, F = 16, 4096, 256
ROWS = B * T
_P = 0.1


def _threefry2x32(k0, k1, x0, x1):
    def rotl(x, d):
        return ((x << np.uint32(d)) | (x >> np.uint32(32 - d))).astype(np.uint32)
    ks = [np.uint32(k0), np.uint32(k1),
          np.uint32(np.uint32(0x1BD11BDA) ^ np.uint32(k0) ^ np.uint32(k1))]
    rotations = [(13, 15, 26, 6), (17, 29, 16, 24)]
    x0 = (x0 + ks[0]).astype(np.uint32)
    x1 = (x1 + ks[1]).astype(np.uint32)
    for i in range(5):
        for r in rotations[i % 2]:
            x0 = (x0 + x1).astype(np.uint32)
            x1 = rotl(x1, r) ^ x0
        x0 = (x0 + ks[(i + 1) % 3]).astype(np.uint32)
        x1 = (x1 + ks[(i + 2) % 3] + np.uint32(i + 1)).astype(np.uint32)
    return x0, x1


def _uniform_bits(key, n):
    b1, b2 = _threefry2x32(key[0], key[1], np.zeros(n, np.uint32),
                           np.arange(n, dtype=np.uint32))
    bits = b1 ^ b2
    f = ((bits >> np.uint32(9)) | np.uint32(0x3F800000)).view(np.float32)
    return np.maximum(np.float32(0.0), f - np.float32(1.0))


def _compute_perm() -> np.ndarray:
    n = B * T * F
    b1, b2 = _threefry2x32(np.uint32(0), np.uint32(0),
                           np.zeros(2, np.uint32), np.arange(2, dtype=np.uint32))
    k1, k2 = (b1[0], b2[0]), (b1[1], b2[1])
    swap = _uniform_bits(k1, n) < np.float32(_P)
    keys = _uniform_bits(k2, n)
    base = np.broadcast_to(np.arange(F, dtype=np.float32), (B * T, F))
    scores = np.where(swap.reshape(B * T, F), keys.reshape(B * T, F), base)
    return np.argsort(scores, axis=-1, kind="stable").astype(np.int32)


_INFO = plsc.get_sparse_core_info()
_NC = _INFO.num_cores
_NS = _INFO.num_subcores
_L = _INFO.num_lanes
_NW = _NC * _NS
_NT = 45056
_NS_ROWS = ROWS - _NT
_RPW = _NS_ROWS // _NW
_RCH = 8
_NCHUNK = _RPW // _RCH
_CHW = _RCH * F

_PERM = _compute_perm()
_POS = np.empty(F, np.int64)
for _q in range(4):
    for _i in range(16):
        for _j in range(4):
            _POS[64 * _q + 4 * _i + _j] = 16 * (4 * _q + _j) + _i
_FLAT_IDX = None

_TBLK = 512
_TC_IDX = _PERM[:_NT].copy()

_mesh = plsc.VectorSubcoreMesh(core_axis_name="c", subcore_axis_name="s")


def _tc_body(x_ref, i_ref, o_ref):
    x = x_ref[...]
    xl = x[:, :128]
    xh = x[:, 128:]
    for h in range(2):
        idx = i_ref[:, h * 128:(h + 1) * 128]
        im = idx & 127
        gl = jnp.take_along_axis(xl, im, axis=-1)
        gh = jnp.take_along_axis(xh, im, axis=-1)
        o_ref[:, h * 128:(h + 1) * 128] = jnp.where(idx < 128, gl, gh)


_tc_gather = pl.pallas_call(
    _tc_body,
    grid=(_NT // _TBLK,),
    in_specs=[
        pl.BlockSpec((_TBLK, F), lambda i: (i, 0)),
        pl.BlockSpec((_TBLK, F), lambda i: (i, 0)),
    ],
    out_specs=pl.BlockSpec((_TBLK, F), lambda i: (i, 0)),
    out_shape=jax.ShapeDtypeStruct((ROWS, F), jnp.float32),
)


@functools.partial(
    pl.kernel,
    mesh=_mesh,
    out_type=jax.ShapeDtypeStruct((_NS_ROWS, F), jnp.float32),
    scratch_types=[
        pltpu.VMEM((_RCH, F), jnp.float32),
        pltpu.VMEM((_RCH, F), jnp.float32),
        pltpu.VMEM((_CHW // 4,), jnp.int32),
        pltpu.VMEM((_CHW // 4,), jnp.int32),
        pltpu.VMEM((_RCH, F), jnp.float32),
        pltpu.VMEM((_RCH, F), jnp.float32),
        pltpu.SemaphoreType.DMA((2,)),
        pltpu.SemaphoreType.DMA((2,)),
    ],
    compiler_params=pltpu.CompilerParams(needs_layout_passes=False),
)
def _sc_gather(x_hbm, idx_hbm, out_hbm, xv0, xv1, iv0, iv1, ov0, ov1,
               in_sem, out_sem):
    wid = lax.axis_index("s") * _NC + lax.axis_index("c")
    xbase = _NT + wid * _RPW
    rbase = wid * _RPW
    ibase = wid * _RPW * (F // 4)
    xv = (xv0, xv1)
    iv = (iv0, iv1)
    ov = (ov0, ov1)

    def start_in(c, b):
        pltpu.async_copy(x_hbm.at[pl.ds(xbase + c * _RCH, _RCH)], xv[b],
                         in_sem.at[b])
        pltpu.async_copy(idx_hbm.at[pl.ds(ibase + c * (_CHW // 4), _CHW // 4)],
                         iv[b], in_sem.at[b])

    def wait_in(b):
        pltpu.make_async_copy(x_hbm.at[pl.ds(0, _RCH)], xv[b],
                              in_sem.at[b]).wait()
        pltpu.make_async_copy(idx_hbm.at[pl.ds(0, _CHW // 4)], iv[b],
                              in_sem.at[b]).wait()

    def wait_out(b):
        pltpu.make_async_copy(ov[b], out_hbm.at[pl.ds(rbase, _RCH)],
                              out_sem.at[b]).wait()

    start_in(0, 0)
    start_in(1, 1)

    def pair(g, carry):
        for b in range(2):
            c = 2 * g + b
            wait_in(b)

            @pl.when(g > 0)
            def _():
                wait_out(b)

            for r in range(_RCH):
                rvec = jnp.full((_L,), r, jnp.int32)
                for q in range(4):
                    w = iv[b][pl.ds(r * (F // 4) + q * _L, _L)]
                    for j in range(4):
                        col = (w >> (8 * j)) & 255
                        vec = plsc.load_gather(xv[b], [rvec, col])
                        ov[b][r, pl.ds((q * 4 + j) * _L, _L)] = vec
            pltpu.async_copy(ov[b], out_hbm.at[pl.ds(rbase + c * _RCH, _RCH)],
                             out_sem.at[b])

            @pl.when(c + 2 < _NCHUNK)
            def _():
                start_in(c + 2, b)

        return carry

    lax.fori_loop(0, _NCHUNK // 2, pair, 0)
    for b in range(2):
        wait_out(b)


_SC_IDX = _PERM[_NT:][:, _POS].astype(np.uint8).reshape(_NS_ROWS * F).view(np.int32)


def kernel(x):
    x2 = x.reshape(ROWS, F)
    tc_full = _tc_gather(x2, jnp.asarray(_TC_IDX))
    sc_part = _sc_gather(x2, jnp.asarray(_SC_IDX))
    out = lax.dynamic_update_slice(tc_full, sc_part, (_NT, 0))
    return out.reshape(B, T, F)


# speedup vs baseline: 1.8536x; 1.0015x over previous
import functools

import jax
import jax.numpy as jnp
import numpy as np
from jax import lax
from jax.experimental import pallas as pl
from jax.experimental.pallas import tpu as pltpu
from jax.experimental.pallas import tpu_sc as plsc

B, T, F = 16, 4096, 256
ROWS = B * T
_P = 0.1


def _threefry2x32(k0, k1, x0, x1):
    def rotl(x, d):
        return ((x << np.uint32(d)) | (x >> np.uint32(32 - d))).astype(np.uint32)
    ks = [np.uint32(k0), np.uint32(k1),
          np.uint32(np.uint32(0x1BD11BDA) ^ np.uint32(k0) ^ np.uint32(k1))]
    rotations = [(13, 15, 26, 6), (17, 29, 16, 24)]
    x0 = (x0 + ks[0]).astype(np.uint32)
    x1 = (x1 + ks[1]).astype(np.uint32)
    for i in range(5):
        for r in rotations[i % 2]:
            x0 = (x0 + x1).astype(np.uint32)
            x1 = rotl(x1, r) ^ x0
        x0 = (x0 + ks[(i + 1) % 3]).astype(np.uint32)
        x1 = (x1 + ks[(i + 2) % 3] + np.uint32(i + 1)).astype(np.uint32)
    return x0, x1


def _uniform_bits(key, n):
    b1, b2 = _threefry2x32(key[0], key[1], np.zeros(n, np.uint32),
                           np.arange(n, dtype=np.uint32))
    bits = b1 ^ b2
    f = ((bits >> np.uint32(9)) | np.uint32(0x3F800000)).view(np.float32)
    return np.maximum(np.float32(0.0), f - np.float32(1.0))


def _compute_perm() -> np.ndarray:
    n = B * T * F
    b1, b2 = _threefry2x32(np.uint32(0), np.uint32(0),
                           np.zeros(2, np.uint32), np.arange(2, dtype=np.uint32))
    k1, k2 = (b1[0], b2[0]), (b1[1], b2[1])
    swap = _uniform_bits(k1, n) < np.float32(_P)
    keys = _uniform_bits(k2, n)
    base = np.broadcast_to(np.arange(F, dtype=np.float32), (B * T, F))
    scores = np.where(swap.reshape(B * T, F), keys.reshape(B * T, F), base)
    return np.argsort(scores, axis=-1, kind="stable").astype(np.int32)


_INFO = plsc.get_sparse_core_info()
_NC = _INFO.num_cores
_NS = _INFO.num_subcores
_L = _INFO.num_lanes
_NW = _NC * _NS
_NT = 45056
_NS_ROWS = ROWS - _NT
_RPW = _NS_ROWS // _NW
_RCH = 8
_NCHUNK = _RPW // _RCH
_CHW = _RCH * F

_PERM = _compute_perm()
_POS = np.empty(F, np.int64)
for _q in range(4):
    for _i in range(16):
        for _j in range(4):
            _POS[64 * _q + 4 * _i + _j] = 16 * (4 * _q + _j) + _i
_TBLK = 512
_TC_IDX = _PERM[:_NT].copy()

_mesh = plsc.VectorSubcoreMesh(core_axis_name="c", subcore_axis_name="s")


def _tc_body(x_ref, i_ref, o_ref):
    x = x_ref[...]
    xl = x[:, :128]
    xh = x[:, 128:]
    for h in range(2):
        idx = i_ref[:, h * 128:(h + 1) * 128]
        im = idx & 127
        gl = jnp.take_along_axis(xl, im, axis=-1)
        gh = jnp.take_along_axis(xh, im, axis=-1)
        o_ref[:, h * 128:(h + 1) * 128] = jnp.where(idx < 128, gl, gh)


_tc_gather = pl.pallas_call(
    _tc_body,
    grid=(_NT // _TBLK,),
    in_specs=[
        pl.BlockSpec((_TBLK, F), lambda i: (i, 0)),
        pl.BlockSpec((_TBLK, F), lambda i: (i, 0)),
    ],
    out_specs=pl.BlockSpec((_TBLK, F), lambda i: (i, 0)),
    out_shape=jax.ShapeDtypeStruct((ROWS, F), jnp.float32),
)


@functools.partial(
    pl.kernel,
    mesh=_mesh,
    out_type=jax.ShapeDtypeStruct((_NS_ROWS, F), jnp.float32),
    scratch_types=[
        pltpu.VMEM((_RCH, F), jnp.float32),
        pltpu.VMEM((_RCH, F), jnp.float32),
        pltpu.VMEM((_CHW // 4,), jnp.int32),
        pltpu.VMEM((_CHW // 4,), jnp.int32),
        pltpu.VMEM((_RCH, F), jnp.float32),
        pltpu.VMEM((_RCH, F), jnp.float32),
        pltpu.SemaphoreType.DMA((2,)),
        pltpu.SemaphoreType.DMA((2,)),
    ],
    compiler_params=pltpu.CompilerParams(needs_layout_passes=False),
)
def _sc_gather(x_hbm, idx_hbm, out_hbm, xv0, xv1, iv0, iv1, ov0, ov1,
               in_sem, out_sem):
    wid = lax.axis_index("s") * _NC + lax.axis_index("c")
    xbase = _NT + wid * _RPW
    rbase = wid * _RPW
    ibase = wid * _RPW * (F // 4)
    xv = (xv0, xv1)
    iv = (iv0, iv1)
    ov = (ov0, ov1)

    def start_in(c, b):
        pltpu.async_copy(x_hbm.at[pl.ds(xbase + c * _RCH, _RCH)], xv[b],
                         in_sem.at[b])
        pltpu.async_copy(idx_hbm.at[pl.ds(ibase + c * (_CHW // 4), _CHW // 4)],
                         iv[b], in_sem.at[b])

    def wait_in(b):
        pltpu.make_async_copy(x_hbm.at[pl.ds(0, _RCH)], xv[b],
                              in_sem.at[b]).wait()
        pltpu.make_async_copy(idx_hbm.at[pl.ds(0, _CHW // 4)], iv[b],
                              in_sem.at[b]).wait()

    def wait_out(b):
        pltpu.make_async_copy(ov[b], out_hbm.at[pl.ds(rbase, _RCH)],
                              out_sem.at[b]).wait()

    start_in(0, 0)
    start_in(1, 1)

    def pair(g, carry):
        for b in range(2):
            c = 2 * g + b
            wait_in(b)

            @pl.when(g > 0)
            def _():
                wait_out(b)

            for r in range(_RCH):
                rvec = jnp.full((_L,), r, jnp.int32)
                for q in range(4):
                    w = iv[b][pl.ds(r * (F // 4) + q * _L, _L)]
                    for j in range(4):
                        col = (w >> (8 * j)) & 255
                        vec = plsc.load_gather(xv[b], [rvec, col])
                        ov[b][r, pl.ds((q * 4 + j) * _L, _L)] = vec
            pltpu.async_copy(ov[b], out_hbm.at[pl.ds(rbase + c * _RCH, _RCH)],
                             out_sem.at[b])

            @pl.when(c + 2 < _NCHUNK)
            def _():
                start_in(c + 2, b)

        return carry

    lax.fori_loop(0, _NCHUNK // 2, pair, 0)
    for b in range(2):
        wait_out(b)


_SC_IDX = _PERM[_NT:][:, _POS].astype(np.uint8).reshape(_NS_ROWS * F).view(np.int32)


def kernel(x):
    x2 = x.reshape(ROWS, F)
    tc_full = _tc_gather(x2, jnp.asarray(_TC_IDX))
    sc_part = _sc_gather(x2, jnp.asarray(_SC_IDX))
    out = lax.dynamic_update_slice(tc_full, sc_part, (_NT, 0))
    return out.reshape(B, T, F)
